# Initial kernel scaffold; baseline (speedup 1.0000x reference)
import jax
import jax.numpy as jnp
from jax.experimental import pallas as pl


def kernel(token_positions, continuous, action, pos_embed, action_embed, W_cont, b_cont, W_out, b_out, gamma, beta):
    raise NotImplementedError("write your pallas kernel here")


# speedup vs baseline: 1.6778x; 1.6778x over previous
import dataclasses
import functools

import jax
import jax.numpy as jnp
from jax import lax
from jax.experimental import pallas as pl
from jax.experimental.pallas import tpu as pltpu
from jax.experimental.pallas import tpu_sc as plsc

B = 16384
D = 128
DW = D // 2
P = 1024
A = 512
NC = 2
NS = 16
NW = NC * NS
EPW = B // NW
CHUNK = 256


def _sc_gather_sum(tok_idx, act_idx, pos_i32, aemb_i32):
    mesh = plsc.VectorSubcoreMesh(core_axis_name="c", subcore_axis_name="s")
    cp = pltpu.CompilerParams()
    if "needs_layout_passes" in pltpu.CompilerParams.__dataclass_fields__:
        cp = dataclasses.replace(cp, needs_layout_passes=False)

    @functools.partial(
        pl.kernel,
        out_type=jax.ShapeDtypeStruct((B * DW,), jnp.int32),
        mesh=mesh,
        compiler_params=cp,
        scratch_types=[
            pltpu.VMEM((P * DW,), jnp.int32),
            pltpu.VMEM((A * DW,), jnp.int32),
            pltpu.VMEM((CHUNK * 8,), jnp.int32),
            pltpu.VMEM((CHUNK,), jnp.int32),
            pltpu.VMEM((CHUNK * DW,), jnp.int32),
        ],
    )
    def k(tok_hbm, aidx_hbm, pos_hbm, aemb_hbm, out_hbm,
          pos_v, aemb_v, tok_v, aidx_v, out_v):
        wid = lax.axis_index("s") * NC + lax.axis_index("c")
        pltpu.sync_copy(pos_hbm, pos_v)
        pltpu.sync_copy(aemb_hbm, aemb_v)
        for chunk in range(EPW // CHUNK):
            base = wid * EPW + chunk * CHUNK
            pltpu.sync_copy(tok_hbm.at[pl.ds(base * 8, CHUNK * 8)], tok_v)
            pltpu.sync_copy(aidx_hbm.at[pl.ds(base, CHUNK)], aidx_v)

            @pl.loop(0, CHUNK, step=16)
            def _(e0):
                lanes = lax.iota(jnp.int32, 16) + e0
                lanes8 = lanes * 8
                abase = plsc.load_gather(aidx_v, [lanes]) * DW
                tbase = [
                    plsc.load_gather(tok_v, [lanes8 + s]) * DW
                    for s in range(8)
                ]
                obase = lanes * DW
                for d in range(DW):
                    acc = plsc.bitcast(
                        plsc.load_gather(aemb_v, [abase + d]), jnp.bfloat16)
                    for s in range(8):
                        acc = acc + plsc.bitcast(
                            plsc.load_gather(pos_v, [tbase[s] + d]), jnp.bfloat16)
                    plsc.store_scatter(out_v, [obase + d],
                                       plsc.bitcast(acc, jnp.int32))

            pltpu.sync_copy(out_v, out_hbm.at[pl.ds(base * DW, CHUNK * DW)])

    return k(tok_idx, act_idx, pos_i32, aemb_i32)


def _tc_tail(emb_bf, cont_pad, wc_pad, b_cont, W_out, b_out, gamma, beta):
    BLK = 1024

    def body(emb_ref, cont_ref, wc_ref, bc_ref, wo_ref, bo_ref, g_ref, bt_ref,
             o_ref):
        emb = emb_ref[...].astype(jnp.float32)
        ce = jnp.dot(cont_ref[...], wc_ref[...],
                     preferred_element_type=jnp.float32) + bc_ref[...]
        turn = (emb + ce) * 0.1
        y = jnp.dot(turn, wo_ref[...],
                    preferred_element_type=jnp.float32) + bo_ref[...]
        m = jnp.mean(y, axis=-1, keepdims=True)
        yc = y - m
        v = jnp.mean(yc * yc, axis=-1, keepdims=True)
        o_ref[...] = yc * lax.rsqrt(v + 1e-5) * g_ref[...] + bt_ref[...]

    full = lambda shape: pl.BlockSpec(shape, lambda i: (0, 0))
    return pl.pallas_call(
        body,
        grid=(B // BLK,),
        in_specs=[
            pl.BlockSpec((BLK, D), lambda i: (i, 0)),
            pl.BlockSpec((BLK, 16), lambda i: (i, 0)),
            full((16, D)),
            full((1, D)),
            full((D, D)),
            full((1, D)),
            full((1, D)),
            full((1, D)),
        ],
        out_specs=pl.BlockSpec((BLK, D), lambda i: (i, 0)),
        out_shape=jax.ShapeDtypeStruct((B, D), jnp.float32),
    )(emb_bf, cont_pad, wc_pad, b_cont, W_out, b_out, gamma, beta)


def kernel(token_positions, continuous, action, pos_embed, action_embed,
           W_cont, b_cont, W_out, b_out, gamma, beta):
    tok_idx = token_positions.astype(jnp.int32).reshape(B * 8)
    act_idx = action.astype(jnp.int32)
    pos_i32 = lax.bitcast_convert_type(
        pos_embed.astype(jnp.bfloat16).reshape(P * DW, 2), jnp.int32)
    aemb_i32 = lax.bitcast_convert_type(
        action_embed.astype(jnp.bfloat16).reshape(A * DW, 2), jnp.int32)
    emb_i32 = _sc_gather_sum(tok_idx, act_idx, pos_i32, aemb_i32)
    emb_bf = lax.bitcast_convert_type(emb_i32, jnp.bfloat16).reshape(B, D)
    cont_pad = jnp.pad(continuous, ((0, 0), (0, 7)))
    wc_pad = jnp.pad(W_cont, ((0, 7), (0, 0)))
    return _tc_tail(emb_bf, cont_pad, wc_pad, b_cont.reshape(1, D), W_out,
                    b_out.reshape(1, D), gamma.reshape(1, D), beta.reshape(1, D))


# speedup vs baseline: 3.0693x; 1.8294x over previous
import dataclasses
import functools

import jax
import jax.numpy as jnp
from jax import lax
from jax.experimental import pallas as pl
from jax.experimental.pallas import tpu as pltpu
from jax.experimental.pallas import tpu_sc as plsc

B = 16384
D = 128
DW = D // 2
P = 1024
A = 512
NC = 2
NS = 16
NW = NC * NS
EPW = B // NW
CHUNK = 256


def _sc_gather_sum(tok_idx, act_idx, pos_i32, aemb_i32):
    mesh = plsc.VectorSubcoreMesh(core_axis_name="c", subcore_axis_name="s")
    cp = pltpu.CompilerParams()
    if "needs_layout_passes" in pltpu.CompilerParams.__dataclass_fields__:
        cp = dataclasses.replace(cp, needs_layout_passes=False)

    @functools.partial(
        pl.kernel,
        out_type=jax.ShapeDtypeStruct((B * DW,), jnp.int32),
        mesh=mesh,
        compiler_params=cp,
        scratch_types=[
            pltpu.VMEM((P * DW,), jnp.int32),
            pltpu.VMEM((A * DW,), jnp.int32),
            pltpu.VMEM((CHUNK * 8,), jnp.int32),
            pltpu.VMEM((CHUNK,), jnp.int32),
            pltpu.VMEM((CHUNK * DW,), jnp.int32),
        ],
    )
    def k(tok_hbm, aidx_hbm, pos_hbm, aemb_hbm, out_hbm,
          pos_v, aemb_v, tok_s, aidx_s, out_v):
        wid = lax.axis_index("s") * NC + lax.axis_index("c")
        pltpu.sync_copy(pos_hbm, pos_v)
        pltpu.sync_copy(aemb_hbm, aemb_v)
        for chunk in range(EPW // CHUNK):
            base = wid * EPW + chunk * CHUNK
            pltpu.sync_copy(tok_hbm.at[pl.ds(base * 8, CHUNK * 8)], tok_s)
            pltpu.sync_copy(aidx_hbm.at[pl.ds(base, CHUNK)], aidx_s)

            @pl.loop(0, CHUNK, step=16)
            def _(g0):
                av = aidx_s[pl.ds(g0, 16)]
                for p in range(8):
                    tv = tok_s[pl.ds(g0 * 8 + p * 16, 16)]
                    for half in range(2):
                        e = g0 + p * 2 + half
                        ab = av[p * 2 + half] * DW
                        acc = [
                            plsc.bitcast(aemb_v[pl.ds(ab + 16 * j, 16)],
                                         jnp.bfloat16)
                            for j in range(4)
                        ]
                        for s in range(8):
                            tb = tv[half * 8 + s] * DW
                            for j in range(4):
                                acc[j] = acc[j] + plsc.bitcast(
                                    pos_v[pl.ds(tb + 16 * j, 16)], jnp.bfloat16)
                        ob = e * DW
                        for j in range(4):
                            out_v[pl.ds(ob + 16 * j, 16)] = plsc.bitcast(
                                acc[j], jnp.int32)

            pltpu.sync_copy(out_v, out_hbm.at[pl.ds(base * DW, CHUNK * DW)])

    return k(tok_idx, act_idx, pos_i32, aemb_i32)


def _tc_tail(emb_bf, cont_pad, wc_pad, b_cont, W_out, b_out, gamma, beta):
    BLK = 1024

    def body(emb_ref, cont_ref, wc_ref, bc_ref, wo_ref, bo_ref, g_ref, bt_ref,
             o_ref):
        emb = emb_ref[...].astype(jnp.float32)
        ce = jnp.dot(cont_ref[...], wc_ref[...],
                     preferred_element_type=jnp.float32) + bc_ref[...]
        turn = (emb + ce) * 0.1
        y = jnp.dot(turn, wo_ref[...],
                    preferred_element_type=jnp.float32) + bo_ref[...]
        m = jnp.mean(y, axis=-1, keepdims=True)
        yc = y - m
        v = jnp.mean(yc * yc, axis=-1, keepdims=True)
        o_ref[...] = yc * lax.rsqrt(v + 1e-5) * g_ref[...] + bt_ref[...]

    full = lambda shape: pl.BlockSpec(shape, lambda i: (0, 0))
    return pl.pallas_call(
        body,
        grid=(B // BLK,),
        in_specs=[
            pl.BlockSpec((BLK, D), lambda i: (i, 0)),
            pl.BlockSpec((BLK, 16), lambda i: (i, 0)),
            full((16, D)),
            full((1, D)),
            full((D, D)),
            full((1, D)),
            full((1, D)),
            full((1, D)),
        ],
        out_specs=pl.BlockSpec((BLK, D), lambda i: (i, 0)),
        out_shape=jax.ShapeDtypeStruct((B, D), jnp.float32),
    )(emb_bf, cont_pad, wc_pad, b_cont, W_out, b_out, gamma, beta)


def kernel(token_positions, continuous, action, pos_embed, action_embed,
           W_cont, b_cont, W_out, b_out, gamma, beta):
    tok_idx = token_positions.astype(jnp.int32).reshape(B * 8)
    act_idx = action.astype(jnp.int32)
    pos_i32 = lax.bitcast_convert_type(
        pos_embed.astype(jnp.bfloat16).reshape(P * DW, 2), jnp.int32)
    aemb_i32 = lax.bitcast_convert_type(
        action_embed.astype(jnp.bfloat16).reshape(A * DW, 2), jnp.int32)
    emb_i32 = _sc_gather_sum(tok_idx, act_idx, pos_i32, aemb_i32)
    emb_bf = lax.bitcast_convert_type(emb_i32, jnp.bfloat16).reshape(B, D)
    cont_pad = jnp.pad(continuous, ((0, 0), (0, 7)))
    wc_pad = jnp.pad(W_cont, ((0, 7), (0, 0)))
    return _tc_tail(emb_bf, cont_pad, wc_pad, b_cont.reshape(1, D), W_out,
                    b_out.reshape(1, D), gamma.reshape(1, D), beta.reshape(1, D))


# speedup vs baseline: 7.4115x; 2.4147x over previous
import dataclasses
import functools

import jax
import jax.numpy as jnp
from jax import lax
from jax.experimental import pallas as pl
from jax.experimental.pallas import tpu as pltpu
from jax.experimental.pallas import tpu_sc as plsc

B = 16384
D = 128
DW = D // 2
P = 1024
A = 512
NC = 2
NS = 16
NW = NC * NS
EPW = B // NW
CHUNK = 128


def _tc_prep(pos_embed, action_embed):
    def _pack(x, rows):
        u = lax.bitcast_convert_type(x.astype(jnp.bfloat16), jnp.uint16)
        lo = u[:, :DW].astype(jnp.uint32)
        hi = u[:, DW:].astype(jnp.uint32)
        w = lo | (hi << 16)
        half = rows // 2
        return lax.bitcast_convert_type(
            jnp.concatenate([w[:half], w[half:]], axis=1), jnp.int32)

    def body(pos_ref, aemb_ref, post_ref, aembt_ref):
        post_ref[...] = _pack(pos_ref[...], P)
        aembt_ref[...] = _pack(aemb_ref[...], A)

    return pl.pallas_call(
        body,
        out_shape=[
            jax.ShapeDtypeStruct((P // 2, D), jnp.int32),
            jax.ShapeDtypeStruct((A // 2, D), jnp.int32),
        ],
    )(pos_embed, action_embed)


def _sc_gather_sum(tok_s, act_idx, post, aembt):
    mesh = plsc.VectorSubcoreMesh(core_axis_name="c", subcore_axis_name="s")
    cp = pltpu.CompilerParams()
    if "needs_layout_passes" in pltpu.CompilerParams.__dataclass_fields__:
        cp = dataclasses.replace(cp, needs_layout_passes=False)

    def _row(tab_v, c, half_rows):
        cw = (c % half_rows) * D + (c // half_rows) * DW
        return [
            plsc.bitcast(tab_v[pl.ds(cw + 16 * j, 16)], jnp.bfloat16)
            for j in range(4)
        ]

    @functools.partial(
        pl.kernel,
        out_type=jax.ShapeDtypeStruct((B * D,), jnp.float32),
        mesh=mesh,
        compiler_params=cp,
        scratch_types=[
            pltpu.VMEM((P * DW,), jnp.int32),
            pltpu.VMEM((A * DW,), jnp.int32),
            pltpu.VMEM((9 * CHUNK,), jnp.int32),
            pltpu.VMEM((CHUNK * D,), jnp.float32),
            pltpu.SemaphoreType.DMA,
        ],
    )
    def k(t0, t1, t2, t3, t4, t5, t6, t7, aidx_hbm, post_hbm, aembt_hbm,
          out_hbm, post_v, aembt_v, idx_v, out_v, sem):
        tok_hbm = [t0, t1, t2, t3, t4, t5, t6, t7]
        wid = lax.axis_index("s") * NC + lax.axis_index("c")
        ct0 = pltpu.async_copy(post_hbm, post_v, sem)
        ct1 = pltpu.async_copy(aembt_hbm, aembt_v, sem)
        for chunk in range(EPW // CHUNK):
            base = wid * EPW + chunk * CHUNK
            cps = [
                pltpu.async_copy(tok_hbm[s].at[pl.ds(base, CHUNK)],
                                 idx_v.at[pl.ds(s * CHUNK, CHUNK)], sem)
                for s in range(8)
            ]
            cps.append(pltpu.async_copy(aidx_hbm.at[pl.ds(base, CHUNK)],
                                        idx_v.at[pl.ds(8 * CHUNK, CHUNK)], sem))
            if chunk == 0:
                ct0.wait()
                ct1.wait()
            for c in cps:
                c.wait()

            @pl.loop(0, CHUNK, step=16)
            def _(g0):
                tv = [idx_v[pl.ds(s * CHUNK + g0, 16)] for s in range(9)]
                for i in range(16):
                    e = g0 + i
                    acc = _row(aembt_v, tv[8][i], A // 2)
                    for s in range(8):
                        row = _row(post_v, tv[s][i], P // 2)
                        for j in range(4):
                            acc[j] = acc[j] + row[j]
                    ew = e * D
                    for j in range(4):
                        flo, fhi = plsc.unpack(
                            acc[j], format=plsc.PackFormat.INTERLEAVED,
                            preferred_element_type=jnp.float32)
                        out_v[pl.ds(ew + 16 * j, 16)] = flo
                        out_v[pl.ds(ew + DW + 16 * j, 16)] = fhi

            pltpu.sync_copy(out_v, out_hbm.at[pl.ds(base * D, CHUNK * D)])

    return k(*tok_s, act_idx, post, aembt)


def _tc_tail(emb, continuous, W_cont, b_cont, W_out, b_out, gamma, beta):
    BLK = 1024

    def body(emb_ref, cont_ref, wc_ref, bc_ref, wo_ref, bo_ref, g_ref, bt_ref,
             o_ref):
        ce = jnp.dot(cont_ref[...], wc_ref[...],
                     preferred_element_type=jnp.float32) + bc_ref[...]
        turn = (emb_ref[...] + ce) * 0.1
        y = jnp.dot(turn, wo_ref[...],
                    preferred_element_type=jnp.float32) + bo_ref[...]
        m = jnp.mean(y, axis=-1, keepdims=True)
        yc = y - m
        v = jnp.mean(yc * yc, axis=-1, keepdims=True)
        o_ref[...] = yc * lax.rsqrt(v + 1e-5) * g_ref[...] + bt_ref[...]

    full = lambda shape: pl.BlockSpec(shape, lambda i: (0, 0))
    return pl.pallas_call(
        body,
        grid=(B // BLK,),
        in_specs=[
            pl.BlockSpec((BLK, D), lambda i: (i, 0)),
            pl.BlockSpec((BLK, 9), lambda i: (i, 0)),
            full((9, D)),
            full((1, D)),
            full((D, D)),
            full((1, D)),
            full((1, D)),
            full((1, D)),
        ],
        out_specs=pl.BlockSpec((BLK, D), lambda i: (i, 0)),
        out_shape=jax.ShapeDtypeStruct((B, D), jnp.float32),
    )(emb, continuous, W_cont, b_cont, W_out, b_out, gamma, beta)


def kernel(token_positions, continuous, action, pos_embed, action_embed,
           W_cont, b_cont, W_out, b_out, gamma, beta):
    tok = token_positions.astype(jnp.int32)
    act_idx = action.astype(jnp.int32)
    tok_s = [tok[:, s] for s in range(8)]
    post, aembt = _tc_prep(pos_embed, action_embed)
    emb_flat = _sc_gather_sum(tok_s, act_idx, post.reshape(P * DW),
                              aembt.reshape(A * DW))
    emb = emb_flat.reshape(B, D)
    return _tc_tail(emb, continuous, W_cont, b_cont.reshape(1, D), W_out,
                    b_out.reshape(1, D), gamma.reshape(1, D), beta.reshape(1, D))
